# Initial kernel scaffold; baseline (speedup 1.0000x reference)
import jax
import jax.numpy as jnp
from jax.experimental import pallas as pl


def kernel(input_bits, connections, memory):
    raise NotImplementedError("write your pallas kernel here")


# speedup vs baseline: 2.0379x; 2.0379x over previous
import functools

import jax
import jax.numpy as jnp
from jax import lax
from jax.experimental import pallas as pl
from jax.experimental.pallas import tpu as pltpu
from jax.experimental.pallas import tpu_sc as plsc

BATCH = 64
TOTAL_INPUT_BITS = 4096
NUM_NEURONS = 8192
N_BITS = 16
HASH_SIZE = 8192

NC, NS, L = 2, 16, 16
NW = NC * NS
NGROUPS = 8
BGROUPS = 4
NPW = NUM_NEURONS // NGROUPS
IDX_ROWS = NPW * L // 128


def _sc_body(bits_hbm, conn_hbm, mem_hbm, out_hbm, bits_v, conn_v, addr_v,
             out_v, sem):
    c = lax.axis_index("c")
    s = lax.axis_index("s")
    wid = s * NC + c
    ng = wid % NGROUPS
    bg = wid // NGROUPS
    n0 = ng * NPW

    pltpu.sync_copy(bits_hbm.at[bg], bits_v)
    pltpu.sync_copy(conn_hbm.at[pl.ds(n0 * N_BITS, NPW * N_BITS)], conn_v)

    iota = lax.iota(jnp.int32, L)

    @pl.loop(0, NPW)
    def _addresses(n):
        acc = jnp.zeros((L,), jnp.int32)
        for i in range(N_BITS):
            nb = jnp.full((L,), n * N_BITS + i, jnp.int32)
            cb = plsc.load_gather(conn_v, [nb])
            bits = plsc.load_gather(bits_v, [cb * L + iota])
            acc = acc + (bits << i)
        addr = (acc & (HASH_SIZE - 1)) + (n0 + n) * HASH_SIZE
        flat = iota * NPW + n
        plsc.store_scatter(addr_v, [flat >> 7, flat & 127], addr)

    @pl.loop(0, IDX_ROWS // 8)
    def _gather(r8):
        cps = [
            pltpu.async_copy(
                mem_hbm.at[addr_v.at[r8 * 8 + k]],
                out_v.at[r8, pl.ds(k * 128, 128)],
                sem,
            )
            for k in range(8)
        ]
        for cp in cps:
            cp.wait()

    @pl.loop(0, L)
    def _writeback(j):
        pltpu.sync_copy(out_v.at[j], out_hbm.at[bg * L + j, pl.ds(n0, NPW)])


@jax.jit
def kernel(input_bits, connections, memory):
    bits_t = input_bits.reshape(BGROUPS, L, TOTAL_INPUT_BITS).transpose(0, 2, 1)
    bits_t = bits_t.astype(jnp.int32).reshape(BGROUPS, TOTAL_INPUT_BITS * L)
    conn_flat = connections.reshape(-1)
    mem_flat = memory.reshape(-1)

    mesh = plsc.VectorSubcoreMesh(
        core_axis_name="c", subcore_axis_name="s", num_cores=NC,
        num_subcores=NS)
    run = pl.kernel(
        _sc_body,
        out_type=jax.ShapeDtypeStruct((BATCH, NUM_NEURONS), jnp.float32),
        mesh=mesh,
        compiler_params=pltpu.CompilerParams(needs_layout_passes=False),
        scratch_types=[
            pltpu.VMEM((TOTAL_INPUT_BITS * L,), jnp.int32),
            pltpu.VMEM((NPW * N_BITS,), jnp.int32),
            pltpu.VMEM((IDX_ROWS, 128), jnp.int32),
            pltpu.VMEM((L, NPW), jnp.float32),
            pltpu.SemaphoreType.DMA,
        ],
    )
    return run(bits_t, conn_flat, mem_flat)


# speedup vs baseline: 8.2138x; 4.0305x over previous
import jax
import jax.numpy as jnp
from jax import lax
from jax.experimental import pallas as pl
from jax.experimental.pallas import tpu as pltpu
from jax.experimental.pallas import tpu_sc as plsc

BATCH = 64
TOTAL_INPUT_BITS = 4096
NUM_NEURONS = 8192
N_BITS = 16
HASH_SIZE = 8192

NC, NS, L = 2, 16, 16
NPW = 512
IDX_ROWS = NPW * 32 // 128


def _sc_body(bits_hbm, conn_hbm, mem_hbm, out_hbm, packin_v, bitsp_v, conn_v,
             addr_v, out_v, shared_q, sem):
    c = lax.axis_index("c")
    s = lax.axis_index("s")
    n0 = s * NPW
    g0 = 2 * c
    rs = s * 256

    iota = lax.iota(jnp.int32, L)

    conn_cp = pltpu.async_copy(
        conn_hbm.at[pl.ds(n0 * N_BITS, NPW * N_BITS)], conn_v, sem)

    with jax.named_scope("pack"):
        pltpu.sync_copy(bits_hbm.at[g0, pl.ds(rs * L, 256 * L)],
                        packin_v.at[pl.ds(0, 256 * L)])
        pltpu.sync_copy(bits_hbm.at[g0 + 1, pl.ds(rs * L, 256 * L)],
                        packin_v.at[pl.ds(256 * L, 256 * L)])

        @pl.loop(0, 256)
        def _pack(r):
            w = packin_v[pl.ds(r * L, L)] | (
                packin_v[pl.ds(256 * L + r * L, L)] << L)
            bitsp_v[pl.ds(rs * L + r * L, L)] = w

        pltpu.sync_copy(bitsp_v.at[pl.ds(rs * L, 256 * L)],
                        shared_q.at[pl.ds(rs * L, 256 * L)])
        plsc.subcore_barrier()
        pltpu.sync_copy(shared_q, bitsp_v)

    conn_cp.wait()

    NCHUNK = 4
    CN = NPW // NCHUNK
    all_cps = []
    for chunk in range(NCHUNK):

        @pl.loop(0, CN)
        def _addresses(nl, chunk=chunk):
            n = chunk * CN + nl
            conn_row = conn_v[pl.ds(n * N_BITS, N_BITS)]
            acc = jnp.zeros((L,), jnp.int32)
            for i in range(13):
                ib = jnp.full((L,), i, jnp.int32)
                cb = conn_row.at[ib].get(mode="promise_in_bounds")
                q = plsc.load_gather(bitsp_v, [cb * L + iota])
                acc = acc + (q << i)
            ngl = n0 + n
            base = (ngl >> 3) * 65536 + (ngl & 7) * 128
            a_lo = acc & (HASH_SIZE - 1)
            a_hi = (acc >> L) & (HASH_SIZE - 1)
            w_lo = (((a_lo >> 7) << 10) | (a_lo & 127)) + base
            w_hi = (((a_hi >> 7) << 10) | (a_hi & 127)) + base
            flat_lo = iota * NPW + n
            flat_hi = (iota + L) * NPW + n
            plsc.store_scatter(addr_v, [flat_lo >> 7, flat_lo & 127], w_lo)
            plsc.store_scatter(addr_v, [flat_hi >> 7, flat_hi & 127], w_hi)

        for k in range(32):
            r = k * NCHUNK + chunk
            all_cps.append(pltpu.async_copy(
                mem_hbm.at[addr_v.at[r]],
                out_v.at[k, pl.ds(chunk * 128, 128)],
                sem,
            ))
    with jax.named_scope("gather_wait"):
        for cp in all_cps:
            cp.wait()

    with jax.named_scope("writeback"):
        @pl.loop(0, 32)
        def _writeback(j):
            pltpu.sync_copy(out_v.at[j],
                            out_hbm.at[c * 32 + j, pl.ds(n0, NPW)])


@jax.jit
def kernel(input_bits, connections, memory):
    bits_t = input_bits.reshape(4, L, TOTAL_INPUT_BITS).transpose(0, 2, 1)
    bits_t = bits_t.astype(jnp.int32).reshape(4, TOTAL_INPUT_BITS * L)
    conn_flat = connections.reshape(-1)
    mem_flat = memory.reshape(1024, 8, 64, 128).transpose(0, 2, 1, 3)
    mem_flat = mem_flat.reshape(-1)

    mesh = plsc.VectorSubcoreMesh(
        core_axis_name="c", subcore_axis_name="s", num_cores=NC,
        num_subcores=NS)
    run = pl.kernel(
        _sc_body,
        out_type=jax.ShapeDtypeStruct((BATCH, NUM_NEURONS), jnp.float32),
        mesh=mesh,
        compiler_params=pltpu.CompilerParams(needs_layout_passes=False),
        scratch_types=[
            pltpu.VMEM((2 * 256 * L,), jnp.int32),
            pltpu.VMEM((TOTAL_INPUT_BITS * L,), jnp.int32),
            pltpu.VMEM((NPW * N_BITS,), jnp.int32),
            pltpu.VMEM((IDX_ROWS, 128), jnp.int32),
            pltpu.VMEM((32, NPW), jnp.float32),
            pltpu.VMEM_SHARED((TOTAL_INPUT_BITS * L,), jnp.int32),
            pltpu.SemaphoreType.DMA,
        ],
    )
    return run(bits_t, conn_flat, mem_flat)
